# baseline (speedup 1.0000x reference)
import functools

import jax
import jax.numpy as jnp
from jax import lax
from jax.experimental import pallas as pl
from jax.experimental.pallas import tpu as pltpu
from jax.experimental.pallas import tpu_sc as plsc

N = 10000
E = 320000
P = 100000
IN_DIM = 128
D = 256
H = 128
NC = 2
NS = 16
CH = 128
NP = 10240
RT = NP // NS
ECH = 160
EG = ECH // 4
EPAD = ECH * NS * CH
PCH = 52
PG = PCH // 4
PPAD = PCH * NS * CH
PCHH = PCH // 2
PPADH = PPAD // 2
DBN = 1024
EC = E // CH
EPS = 1e-5
BN = 1000

@functools.lru_cache(maxsize=None)
def _mesh():
    return plsc.VectorSubcoreMesh(core_axis_name="c", subcore_axis_name="s",
                                  num_cores=NC, num_subcores=NS)


def _bdot(a, w):
    return jnp.dot(a.astype(jnp.bfloat16), w.astype(jnp.bfloat16),
                   preferred_element_type=jnp.float32)


def _ln(t, g, b):
    mu = jnp.mean(t, axis=-1, keepdims=True)
    var = jnp.mean((t - mu) ** 2, axis=-1, keepdims=True)
    return (t - mu) * lax.rsqrt(var + EPS) * g + b


def _enc_body(x_ref, w_ref, b_ref, g_ref, be_ref, o_ref):
    t = _bdot(x_ref[...], w_ref[...])
    t = jnp.maximum(_ln(t + b_ref[...], g_ref[...], be_ref[...]), 0.0)
    o_ref[0] = t[:, :H]
    o_ref[1] = t[:, H:]


def _encode(x, W_in, b_in, g_in, be_in):
    return pl.pallas_call(
        _enc_body,
        grid=(N // BN,),
        in_specs=[
            pl.BlockSpec((BN, IN_DIM), lambda i: (i, 0)),
            pl.BlockSpec((IN_DIM, D), lambda i: (0, 0)),
            pl.BlockSpec((1, D), lambda i: (0, 0)),
            pl.BlockSpec((1, D), lambda i: (0, 0)),
            pl.BlockSpec((1, D), lambda i: (0, 0)),
        ],
        out_specs=pl.BlockSpec((2, BN, H), lambda i: (0, i, 0)),
        out_shape=jax.ShapeDtypeStruct((2, N, H), jnp.float32),
    )(x, W_in, b_in.reshape(1, D), g_in.reshape(1, D), be_in.reshape(1, D))


def _convA_body(h_ref, wr_ref, bl_ref, o_ref):
    h = jnp.concatenate([h_ref[0], h_ref[1]], axis=1)
    t = _bdot(h, wr_ref[...]) + bl_ref[...]
    o_ref[0] = t[:, :H]
    o_ref[1] = t[:, H:]


def _convA(h2, Wr, bl):
    return pl.pallas_call(
        _convA_body,
        grid=(N // BN,),
        in_specs=[
            pl.BlockSpec((2, BN, H), lambda i: (0, i, 0)),
            pl.BlockSpec((D, D), lambda i: (0, 0)),
            pl.BlockSpec((1, D), lambda i: (0, 0)),
        ],
        out_specs=pl.BlockSpec((2, BN, H), lambda i: (0, i, 0)),
        out_shape=jax.ShapeDtypeStruct((2, N, H), jnp.float32),
    )(h2, Wr, bl.reshape(1, D))


def _convB_body(h_ref, a_ref, c_ref, hw_ref, wl_ref, g_ref, b_ref,
                o_ref, *, residual):
    h = jnp.concatenate([h_ref[0], h_ref[1]], axis=1)
    agg = jnp.concatenate([a_ref[0], a_ref[1]], axis=1)
    hwr = jnp.concatenate([hw_ref[0], hw_ref[1]], axis=1)
    cnt = jnp.maximum(c_ref[0, :, 0:1] + c_ref[1, :, 0:1], 1.0)
    t = _bdot(agg / cnt, wl_ref[...]) + hwr
    t = jnp.maximum(_ln(t, g_ref[...], b_ref[...]), 0.0)
    if residual:
        t = t + h
    o_ref[0] = t[:, :H]
    o_ref[1] = t[:, H:]


def _convB(h2, agg2, cnt2, hwr2, Wl, gn, bn, residual):
    return pl.pallas_call(
        functools.partial(_convB_body, residual=residual),
        grid=(N // BN,),
        in_specs=[
            pl.BlockSpec((2, BN, H), lambda i: (0, i, 0)),
            pl.BlockSpec((2, BN, H), lambda i: (0, i, 0)),
            pl.BlockSpec((2, BN, H), lambda i: (0, i, 0)),
            pl.BlockSpec((2, BN, H), lambda i: (0, i, 0)),
            pl.BlockSpec((D, D), lambda i: (0, 0)),
            pl.BlockSpec((1, D), lambda i: (0, 0)),
            pl.BlockSpec((1, D), lambda i: (0, 0)),
        ],
        out_specs=pl.BlockSpec((2, BN, H), lambda i: (0, i, 0)),
        out_shape=jax.ShapeDtypeStruct((2, N, H), jnp.float32),
    )(h2, agg2, cnt2, hwr2, Wl, gn.reshape(1, D), bn.reshape(1, D))


def _dec_body(u_ref, v_ref, w0_ref, b0_ref, g0_ref, be0_ref,
              w1_ref, b1_ref, g1_ref, be1_ref, w2_ref, b2_ref, o_ref):
    zu = jnp.concatenate([u_ref[0], u_ref[1]], axis=1)
    zv = jnp.concatenate([v_ref[0], v_ref[1]], axis=1)
    t = _bdot(zu * zv, w0_ref[...])
    t = jnp.maximum(_ln(t + b0_ref[...], g0_ref[...], be0_ref[...]), 0.0)
    t = _bdot(t, w1_ref[...])
    t = jnp.maximum(_ln(t + b1_ref[...], g1_ref[...], be1_ref[...]), 0.0)
    o_ref[...] = _bdot(t, w2_ref[...]) + b2_ref[...]


def _decode(zu2, zv2, Wd0, bd0, gd0, bed0, Wd1, bd1, gd1, bed1, Wd2, bd2):
    return pl.pallas_call(
        _dec_body,
        grid=(PPADH // DBN,),
        in_specs=[
            pl.BlockSpec((2, DBN, H), lambda i: (0, i, 0)),
            pl.BlockSpec((2, DBN, H), lambda i: (0, i, 0)),
            pl.BlockSpec((D, D), lambda i: (0, 0)),
            pl.BlockSpec((1, D), lambda i: (0, 0)),
            pl.BlockSpec((1, D), lambda i: (0, 0)),
            pl.BlockSpec((1, D), lambda i: (0, 0)),
            pl.BlockSpec((D, D), lambda i: (0, 0)),
            pl.BlockSpec((1, D), lambda i: (0, 0)),
            pl.BlockSpec((1, D), lambda i: (0, 0)),
            pl.BlockSpec((1, D), lambda i: (0, 0)),
            pl.BlockSpec((D, 1), lambda i: (0, 0)),
            pl.BlockSpec((1, 1), lambda i: (0, 0)),
        ],
        out_specs=pl.BlockSpec((DBN, 1), lambda i: (i, 0)),
        out_shape=jax.ShapeDtypeStruct((PPADH, 1), jnp.float32),
    )(zu2, zv2, Wd0, bd0.reshape(1, D), gd0.reshape(1, D), bed0.reshape(1, D),
      Wd1, bd1.reshape(1, D), gd1.reshape(1, D), bed1.reshape(1, D),
      Wd2, bd2.reshape(1, 1))


def _fill_rows(buf, nrows, width, value):
    def body(i, carry):
        for j in range(width // 16):
            buf[i, pl.ds(j * 16, 16)] = jnp.full((16,), value, jnp.float32)
        return carry
    lax.fori_loop(0, nrows, body, 0)


def _agg_body(h2_hbm, src2_hbm, dst2_hbm, out_hbm, idx_s, idx_d, rows,
              acc_sh, sem):
    c = lax.axis_index("c")
    s = lax.axis_index("s")
    _fill_rows(rows, CH, H, 0.0)
    for k in range(RT // CH):
        pltpu.sync_copy(rows, acc_sh.at[pl.ds(s * RT + k * CH, CH)])
    plsc.subcore_barrier()

    def step(i, carry):
        ch = i * NS + s
        pltpu.sync_copy(src2_hbm.at[c, pl.ds(ch * CH, CH)], idx_s)
        pltpu.sync_copy(dst2_hbm.at[pl.ds(ch * CH, CH)], idx_d)
        pltpu.async_copy(h2_hbm.at[idx_s], rows, sem).wait()
        pltpu.sync_copy(rows, acc_sh.at[idx_d], add=True)
        return carry

    lax.fori_loop(0, ECH, step, 0)
    plsc.subcore_barrier()
    pltpu.sync_copy(acc_sh.at[pl.ds(s * RT, RT)],
                    out_hbm.at[pl.ds(c * NP + s * RT, RT)])


@functools.lru_cache(maxsize=None)
def _sc_agg():
    return pl.kernel(
        _agg_body,
        out_type=jax.ShapeDtypeStruct((2 * NP, H), jnp.float32),
        mesh=_mesh(),
        scratch_types=[
            pltpu.VMEM((CH,), jnp.int32),
            pltpu.VMEM((CH,), jnp.int32),
            pltpu.VMEM((CH, H), jnp.float32),
            pltpu.VMEM_SHARED((NP, H), jnp.float32),
            pltpu.SemaphoreType.DMA,
        ],
    )


def _cnt_body(dst_hbm, out_hbm, idx_d, ones_v, cnt_sh):
    c = lax.axis_index("c")
    s = lax.axis_index("s")
    wid = s * NC + c
    _fill_rows(ones_v, CH, H, 0.0)
    for k in range(RT // CH):
        pltpu.sync_copy(ones_v, cnt_sh.at[pl.ds(s * RT + k * CH, CH)])
    _fill_rows(ones_v, CH, H, 1.0)
    plsc.subcore_barrier()

    def step(i, carry):
        ch = i * (NC * NS) + wid

        @pl.when(ch < EC)
        def _():
            pltpu.sync_copy(dst_hbm.at[pl.ds(ch * CH, CH)], idx_d)
            pltpu.sync_copy(ones_v, cnt_sh.at[idx_d], add=True)
        return carry

    lax.fori_loop(0, (EC + NC * NS - 1) // (NC * NS), step, 0)
    plsc.subcore_barrier()
    pltpu.sync_copy(cnt_sh.at[pl.ds(s * RT, RT)],
                    out_hbm.at[pl.ds(c * NP + s * RT, RT)])


@functools.lru_cache(maxsize=None)
def _sc_counts():
    return pl.kernel(
        _cnt_body,
        out_type=jax.ShapeDtypeStruct((2 * NP, H), jnp.float32),
        mesh=_mesh(),
        scratch_types=[
            pltpu.VMEM((CH,), jnp.int32),
            pltpu.VMEM((CH, H), jnp.float32),
            pltpu.VMEM_SHARED((NP, H), jnp.float32),
        ],
    )


def _pair_body(z2_hbm, u2_hbm, v2_hbm, zu_hbm, zv_hbm, idx_u, idx_v,
               rows_u, rows_v, sem_u, sem_v, *, part):
    c = lax.axis_index("c")
    s = lax.axis_index("s")

    def step(i, carry):
        chl = i * NS + s
        ch = part * (PCHH * NS) + chl
        pltpu.sync_copy(u2_hbm.at[c, pl.ds(ch * CH, CH)], idx_u)
        pltpu.sync_copy(v2_hbm.at[c, pl.ds(ch * CH, CH)], idx_v)
        cu = pltpu.async_copy(z2_hbm.at[idx_u], rows_u, sem_u)
        cv = pltpu.async_copy(z2_hbm.at[idx_v], rows_v, sem_v)
        cu.wait()
        cv.wait()
        pltpu.sync_copy(rows_u, zu_hbm.at[pl.ds(c * PPADH + chl * CH, CH)])
        pltpu.sync_copy(rows_v, zv_hbm.at[pl.ds(c * PPADH + chl * CH, CH)])
        return carry

    lax.fori_loop(0, PCHH, step, 0)


@functools.lru_cache(maxsize=None)
def _sc_pair(part):
    return pl.kernel(
        functools.partial(_pair_body, part=part),
        out_type=(jax.ShapeDtypeStruct((2 * PPADH, H), jnp.float32),
                  jax.ShapeDtypeStruct((2 * PPADH, H), jnp.float32)),
        mesh=_mesh(),
        scratch_types=[
            pltpu.VMEM((CH,), jnp.int32),
            pltpu.VMEM((CH,), jnp.int32),
            pltpu.VMEM((CH, H), jnp.float32),
            pltpu.VMEM((CH, H), jnp.float32),
            pltpu.SemaphoreType.DMA,
            pltpu.SemaphoreType.DMA,
        ],
    )


def kernel(x, edge_index, edge_pairs, W_in, b_in, g_in, be_in,
           Wl0, bl0, Wr0, gn0, bn0,
           Wl1, bl1, Wr1, gn1, bn1,
           Wl2, bl2, Wr2, gn2, bn2,
           Wd0, bd0, gd0, bed0, Wd1, bd1, gd1, bed1, Wd2, bd2):
    src = edge_index[0]
    dst = edge_index[1]
    epi = jnp.arange(EPAD - E, dtype=jnp.int32)
    srcp = jnp.concatenate([src, epi % N])
    dstp = jnp.concatenate([dst, N + epi % (NP - N)])
    src2 = jnp.stack([srcp, srcp + N])
    dst2 = dstp
    u = edge_pairs[:, 0]
    v = edge_pairs[:, 1]
    ppi = jnp.arange(PPAD - P, dtype=jnp.int32) % N
    upad = jnp.concatenate([u, ppi])
    vpad = jnp.concatenate([v, ppi])
    u2 = jnp.stack([upad, upad + N])
    v2 = jnp.stack([vpad, vpad + N])

    cnt2 = _sc_counts()(dst).reshape(2, NP, H)
    h2 = _encode(x, W_in, b_in, g_in, be_in)

    convs = [(Wl0, bl0, Wr0, gn0, bn0),
             (Wl1, bl1, Wr1, gn1, bn1),
             (Wl2, bl2, Wr2, gn2, bn2)]
    for i, (Wl, bl, Wr, gn, bn) in enumerate(convs):
        hwr = _convA(h2, Wr, bl)
        agg = _sc_agg()(h2.reshape(2 * N, H), src2, dst2)
        h2 = _convB(h2, agg.reshape(2, NP, H), cnt2, hwr, Wl, gn, bn,
                    residual=(i > 0))

    z2f = h2.reshape(2 * N, H)
    zu0, zv0 = _sc_pair(0)(z2f, u2, v2)
    zu1, zv1 = _sc_pair(1)(z2f, u2, v2)
    dec_w = (Wd0, bd0, gd0, bed0, Wd1, bd1, gd1, bed1, Wd2, bd2)
    out0 = _decode(zu0.reshape(2, PPADH, H), zv0.reshape(2, PPADH, H), *dec_w)
    out1 = _decode(zu1.reshape(2, PPADH, H), zv1.reshape(2, PPADH, H), *dec_w)
    return jnp.concatenate([out0[:, 0], out1[:, 0]])[:P]


# speedup vs baseline: 1.0043x; 1.0043x over previous
import functools

import jax
import jax.numpy as jnp
from jax import lax
from jax.experimental import pallas as pl
from jax.experimental.pallas import tpu as pltpu
from jax.experimental.pallas import tpu_sc as plsc

N = 10000
E = 320000
P = 100000
IN_DIM = 128
D = 256
H = 128
NC = 2
NS = 16
CH = 128
NP = 10240
RT = NP // NS
ECH = 160
EG = ECH // 4
EPAD = ECH * NS * CH
PCH = 52
PG = PCH // 4
PPAD = PCH * NS * CH
PCHH = PCH // 2
PPADH = PPAD // 2
DBN = 1024
EC = E // CH
EPS = 1e-5
BN = 1000

@functools.lru_cache(maxsize=None)
def _mesh():
    return plsc.VectorSubcoreMesh(core_axis_name="c", subcore_axis_name="s",
                                  num_cores=NC, num_subcores=NS)


def _bdot(a, w):
    return jnp.dot(a.astype(jnp.bfloat16), w.astype(jnp.bfloat16),
                   preferred_element_type=jnp.float32)


def _ln(t, g, b):
    mu = jnp.mean(t, axis=-1, keepdims=True)
    var = jnp.mean((t - mu) ** 2, axis=-1, keepdims=True)
    return (t - mu) * lax.rsqrt(var + EPS) * g + b


def _enc_body(x_ref, w_ref, b_ref, g_ref, be_ref, o_ref):
    t = _bdot(x_ref[...], w_ref[...])
    t = jnp.maximum(_ln(t + b_ref[...], g_ref[...], be_ref[...]), 0.0)
    o_ref[0] = t[:, :H]
    o_ref[1] = t[:, H:]


def _encode(x, W_in, b_in, g_in, be_in):
    return pl.pallas_call(
        _enc_body,
        grid=(N // BN,),
        in_specs=[
            pl.BlockSpec((BN, IN_DIM), lambda i: (i, 0)),
            pl.BlockSpec((IN_DIM, D), lambda i: (0, 0)),
            pl.BlockSpec((1, D), lambda i: (0, 0)),
            pl.BlockSpec((1, D), lambda i: (0, 0)),
            pl.BlockSpec((1, D), lambda i: (0, 0)),
        ],
        out_specs=pl.BlockSpec((2, BN, H), lambda i: (0, i, 0)),
        out_shape=jax.ShapeDtypeStruct((2, N, H), jnp.float32),
    )(x, W_in, b_in.reshape(1, D), g_in.reshape(1, D), be_in.reshape(1, D))


def _conv_body(h_ref, a_ref, c_ref, wl_ref, bl_ref, wr_ref, g_ref, b_ref,
               o_ref, *, residual):
    h = jnp.concatenate([h_ref[0], h_ref[1]], axis=1)
    agg = jnp.concatenate([a_ref[0], a_ref[1]], axis=1)
    cnt = jnp.maximum(c_ref[0, :, 0:1] + c_ref[1, :, 0:1], 1.0)
    t = _bdot(agg / cnt, wl_ref[...]) + bl_ref[...] + _bdot(h, wr_ref[...])
    t = jnp.maximum(_ln(t, g_ref[...], b_ref[...]), 0.0)
    if residual:
        t = t + h
    o_ref[0] = t[:, :H]
    o_ref[1] = t[:, H:]


def _conv(h2, agg2, cnt2, Wl, bl, Wr, gn, bn, residual):
    return pl.pallas_call(
        functools.partial(_conv_body, residual=residual),
        grid=(N // BN,),
        in_specs=[
            pl.BlockSpec((2, BN, H), lambda i: (0, i, 0)),
            pl.BlockSpec((2, BN, H), lambda i: (0, i, 0)),
            pl.BlockSpec((2, BN, H), lambda i: (0, i, 0)),
            pl.BlockSpec((D, D), lambda i: (0, 0)),
            pl.BlockSpec((1, D), lambda i: (0, 0)),
            pl.BlockSpec((D, D), lambda i: (0, 0)),
            pl.BlockSpec((1, D), lambda i: (0, 0)),
            pl.BlockSpec((1, D), lambda i: (0, 0)),
        ],
        out_specs=pl.BlockSpec((2, BN, H), lambda i: (0, i, 0)),
        out_shape=jax.ShapeDtypeStruct((2, N, H), jnp.float32),
    )(h2, agg2, cnt2, Wl, bl.reshape(1, D), Wr, gn.reshape(1, D),
      bn.reshape(1, D))


def _dec_body(u_ref, v_ref, w0_ref, b0_ref, g0_ref, be0_ref,
              w1_ref, b1_ref, g1_ref, be1_ref, w2_ref, b2_ref, o_ref):
    zu = jnp.concatenate([u_ref[0], u_ref[1]], axis=1)
    zv = jnp.concatenate([v_ref[0], v_ref[1]], axis=1)
    t = _bdot(zu * zv, w0_ref[...])
    t = jnp.maximum(_ln(t + b0_ref[...], g0_ref[...], be0_ref[...]), 0.0)
    t = _bdot(t, w1_ref[...])
    t = jnp.maximum(_ln(t + b1_ref[...], g1_ref[...], be1_ref[...]), 0.0)
    o_ref[...] = _bdot(t, w2_ref[...]) + b2_ref[...]


def _decode(zu2, zv2, Wd0, bd0, gd0, bed0, Wd1, bd1, gd1, bed1, Wd2, bd2):
    return pl.pallas_call(
        _dec_body,
        grid=(PPADH // DBN,),
        in_specs=[
            pl.BlockSpec((2, DBN, H), lambda i: (0, i, 0)),
            pl.BlockSpec((2, DBN, H), lambda i: (0, i, 0)),
            pl.BlockSpec((D, D), lambda i: (0, 0)),
            pl.BlockSpec((1, D), lambda i: (0, 0)),
            pl.BlockSpec((1, D), lambda i: (0, 0)),
            pl.BlockSpec((1, D), lambda i: (0, 0)),
            pl.BlockSpec((D, D), lambda i: (0, 0)),
            pl.BlockSpec((1, D), lambda i: (0, 0)),
            pl.BlockSpec((1, D), lambda i: (0, 0)),
            pl.BlockSpec((1, D), lambda i: (0, 0)),
            pl.BlockSpec((D, 1), lambda i: (0, 0)),
            pl.BlockSpec((1, 1), lambda i: (0, 0)),
        ],
        out_specs=pl.BlockSpec((DBN, 1), lambda i: (i, 0)),
        out_shape=jax.ShapeDtypeStruct((PPADH, 1), jnp.float32),
    )(zu2, zv2, Wd0, bd0.reshape(1, D), gd0.reshape(1, D), bed0.reshape(1, D),
      Wd1, bd1.reshape(1, D), gd1.reshape(1, D), bed1.reshape(1, D),
      Wd2, bd2.reshape(1, 1))


def _fill_rows(buf, nrows, width, value):
    def body(i, carry):
        for j in range(width // 16):
            buf[i, pl.ds(j * 16, 16)] = jnp.full((16,), value, jnp.float32)
        return carry
    lax.fori_loop(0, nrows, body, 0)


def _agg_body(h2_hbm, src2_hbm, dst2_hbm, out_hbm, idx_s, idx_d, rows,
              acc_sh, sem):
    c = lax.axis_index("c")
    s = lax.axis_index("s")
    _fill_rows(rows, CH, H, 0.0)
    for k in range(RT // CH):
        pltpu.sync_copy(rows, acc_sh.at[pl.ds(s * RT + k * CH, CH)])
    plsc.subcore_barrier()

    def step(i, carry):
        ch = i * NS + s
        pltpu.sync_copy(src2_hbm.at[c, pl.ds(ch * CH, CH)], idx_s)
        pltpu.sync_copy(dst2_hbm.at[pl.ds(ch * CH, CH)], idx_d)
        pltpu.async_copy(h2_hbm.at[idx_s], rows, sem).wait()
        pltpu.sync_copy(rows, acc_sh.at[idx_d], add=True)
        return carry

    lax.fori_loop(0, ECH, step, 0)
    plsc.subcore_barrier()
    pltpu.sync_copy(acc_sh.at[pl.ds(s * RT, RT)],
                    out_hbm.at[pl.ds(c * NP + s * RT, RT)])


@functools.lru_cache(maxsize=None)
def _sc_agg():
    return pl.kernel(
        _agg_body,
        out_type=jax.ShapeDtypeStruct((2 * NP, H), jnp.float32),
        mesh=_mesh(),
        scratch_types=[
            pltpu.VMEM((CH,), jnp.int32),
            pltpu.VMEM((CH,), jnp.int32),
            pltpu.VMEM((CH, H), jnp.float32),
            pltpu.VMEM_SHARED((NP, H), jnp.float32),
            pltpu.SemaphoreType.DMA,
        ],
    )


def _cnt_body(dst_hbm, out_hbm, idx_d, ones_v, cnt_sh):
    c = lax.axis_index("c")
    s = lax.axis_index("s")
    wid = s * NC + c
    _fill_rows(ones_v, CH, H, 0.0)
    for k in range(RT // CH):
        pltpu.sync_copy(ones_v, cnt_sh.at[pl.ds(s * RT + k * CH, CH)])
    _fill_rows(ones_v, CH, H, 1.0)
    plsc.subcore_barrier()

    def step(i, carry):
        ch = i * (NC * NS) + wid

        @pl.when(ch < EC)
        def _():
            pltpu.sync_copy(dst_hbm.at[pl.ds(ch * CH, CH)], idx_d)
            pltpu.sync_copy(ones_v, cnt_sh.at[idx_d], add=True)
        return carry

    lax.fori_loop(0, (EC + NC * NS - 1) // (NC * NS), step, 0)
    plsc.subcore_barrier()
    pltpu.sync_copy(cnt_sh.at[pl.ds(s * RT, RT)],
                    out_hbm.at[pl.ds(c * NP + s * RT, RT)])


@functools.lru_cache(maxsize=None)
def _sc_counts():
    return pl.kernel(
        _cnt_body,
        out_type=jax.ShapeDtypeStruct((2 * NP, H), jnp.float32),
        mesh=_mesh(),
        scratch_types=[
            pltpu.VMEM((CH,), jnp.int32),
            pltpu.VMEM((CH, H), jnp.float32),
            pltpu.VMEM_SHARED((NP, H), jnp.float32),
        ],
    )


def _pair_body(z2_hbm, u2_hbm, v2_hbm, zu_hbm, zv_hbm, idx_u, idx_v,
               rows_u, rows_v, sem_u, sem_v, *, part):
    c = lax.axis_index("c")
    s = lax.axis_index("s")

    def step(i, carry):
        chl = i * NS + s
        ch = part * (PCHH * NS) + chl
        pltpu.sync_copy(u2_hbm.at[c, pl.ds(ch * CH, CH)], idx_u)
        pltpu.sync_copy(v2_hbm.at[c, pl.ds(ch * CH, CH)], idx_v)
        cu = pltpu.async_copy(z2_hbm.at[idx_u], rows_u, sem_u)
        cv = pltpu.async_copy(z2_hbm.at[idx_v], rows_v, sem_v)
        cu.wait()
        cv.wait()
        pltpu.sync_copy(rows_u, zu_hbm.at[pl.ds(c * PPADH + chl * CH, CH)])
        pltpu.sync_copy(rows_v, zv_hbm.at[pl.ds(c * PPADH + chl * CH, CH)])
        return carry

    lax.fori_loop(0, PCHH, step, 0)


@functools.lru_cache(maxsize=None)
def _sc_pair(part):
    return pl.kernel(
        functools.partial(_pair_body, part=part),
        out_type=(jax.ShapeDtypeStruct((2 * PPADH, H), jnp.float32),
                  jax.ShapeDtypeStruct((2 * PPADH, H), jnp.float32)),
        mesh=_mesh(),
        scratch_types=[
            pltpu.VMEM((CH,), jnp.int32),
            pltpu.VMEM((CH,), jnp.int32),
            pltpu.VMEM((CH, H), jnp.float32),
            pltpu.VMEM((CH, H), jnp.float32),
            pltpu.SemaphoreType.DMA,
            pltpu.SemaphoreType.DMA,
        ],
    )


def kernel(x, edge_index, edge_pairs, W_in, b_in, g_in, be_in,
           Wl0, bl0, Wr0, gn0, bn0,
           Wl1, bl1, Wr1, gn1, bn1,
           Wl2, bl2, Wr2, gn2, bn2,
           Wd0, bd0, gd0, bed0, Wd1, bd1, gd1, bed1, Wd2, bd2):
    src = edge_index[0]
    dst = edge_index[1]
    epi = jnp.arange(EPAD - E, dtype=jnp.int32)
    srcp = jnp.concatenate([src, epi % N])
    dstp = jnp.concatenate([dst, N + epi % (NP - N)])
    src2 = jnp.stack([srcp, srcp + N])
    dst2 = dstp
    u = edge_pairs[:, 0]
    v = edge_pairs[:, 1]
    ppi = jnp.arange(PPAD - P, dtype=jnp.int32) % N
    upad = jnp.concatenate([u, ppi])
    vpad = jnp.concatenate([v, ppi])
    u2 = jnp.stack([upad, upad + N])
    v2 = jnp.stack([vpad, vpad + N])

    cnt2 = _sc_counts()(dst).reshape(2, NP, H)
    h2 = _encode(x, W_in, b_in, g_in, be_in)

    convs = [(Wl0, bl0, Wr0, gn0, bn0),
             (Wl1, bl1, Wr1, gn1, bn1),
             (Wl2, bl2, Wr2, gn2, bn2)]
    for i, (Wl, bl, Wr, gn, bn) in enumerate(convs):
        agg = _sc_agg()(h2.reshape(2 * N, H), src2, dst2)
        h2 = _conv(h2, agg.reshape(2, NP, H), cnt2, Wl, bl, Wr, gn, bn,
                   residual=(i > 0))

    z2f = h2.reshape(2 * N, H)
    zu0, zv0 = _sc_pair(0)(z2f, u2, v2)
    zu1, zv1 = _sc_pair(1)(z2f, u2, v2)
    dec_w = (Wd0, bd0, gd0, bed0, Wd1, bd1, gd1, bed1, Wd2, bd2)
    out0 = _decode(zu0.reshape(2, PPADH, H), zv0.reshape(2, PPADH, H), *dec_w)
    out1 = _decode(zu1.reshape(2, PPADH, H), zv1.reshape(2, PPADH, H), *dec_w)
    return jnp.concatenate([out0[:, 0], out1[:, 0]])[:P]
